# baseline (device time: 251046 ns/iter reference)
import jax
import jax.numpy as jnp
from jax import lax
from jax.experimental import pallas as pl
from jax.experimental.pallas import tpu as pltpu

_S = 256
_C = 2048
_TILE = 128


def kernel(partial, resid, gamma):
    m, d = resid.shape
    p = partial.reshape(m, d)
    g2 = gamma.reshape(1, -1)

    def body(
        p_hbm, r_hbm, g_ref, out_ref,
        pst, pstb, ppr, rst,
        loc_sems, x_send, x_recv,
        p1a_send, p1a_recv, p1b_send, p1b_recv,
        p2a_send, p2a_recv, p2b_send, p2b_recv,
    ):
        X = lax.axis_index("x")
        Y = lax.axis_index("y")
        Z = lax.axis_index("z")
        s = 4 * Y + Z
        r0 = s * _S

        cp_p = pltpu.make_async_copy(
            p_hbm.at[pl.ds(r0, _S), :], pst, loc_sems.at[0]
        )
        cp_r = pltpu.make_async_copy(
            r_hbm.at[pl.ds(r0, _S), :], rst, loc_sems.at[1]
        )
        cp_p.start()
        cp_r.start()

        barrier = pltpu.get_barrier_semaphore()

        def sig(dev):
            pl.semaphore_signal(
                barrier, inc=1, device_id=dev,
                device_id_type=pl.DeviceIdType.MESH,
            )

        sig((1 - X, Y, Z))
        pl.when(Y > 0)(lambda: sig((X, Y - 1, Z)))
        pl.when(Y < 3)(lambda: sig((X, Y + 1, Z)))
        pl.when(Z > 0)(lambda: sig((X, Y, Z - 1)))
        pl.when(Z < 3)(lambda: sig((X, Y, Z + 1)))
        pl.semaphore_wait(barrier, 1)
        pl.when(Y > 0)(lambda: pl.semaphore_wait(barrier, 1))
        pl.when(Y < 3)(lambda: pl.semaphore_wait(barrier, 1))
        pl.when(Z > 0)(lambda: pl.semaphore_wait(barrier, 1))
        pl.when(Z < 3)(lambda: pl.semaphore_wait(barrier, 1))

        cp_p.wait()
        pstb[:, :] = pst[:, :].astype(jnp.bfloat16)
        xr = pltpu.make_async_remote_copy(
            src_ref=pstb,
            dst_ref=ppr,
            send_sem=x_send,
            recv_sem=x_recv,
            device_id=(1 - X, Y, Z),
            device_id_type=pl.DeviceIdType.MESH,
        )
        xr.start()
        xr.wait_recv()
        cp_r.wait()

        g = g_ref[:, :].astype(jnp.float32)
        for t in range(_S // _TILE):
            tr = pl.ds(t * _TILE, _TILE)
            y32 = (
                pst[tr, :]
                + ppr[tr, :].astype(jnp.float32)
                + rst[tr, :]
            )
            ms = jnp.mean(y32 * y32, axis=-1, keepdims=True)
            out_ref[pl.ds(r0 + t * _TILE, _TILE), :] = (
                y32 * lax.rsqrt(ms + 1e-6) * g
            ).astype(jnp.bfloat16)

        started = []

        def mkphase(axis, col0, chunks_fn, send_sems, recv_sems, nchunk):
            pos = Z if axis == "z" else Y

            def dev(off):
                if axis == "z":
                    return (X, Y, Z + off)
                return (X, Y + off, Z)

            def ops(di):
                if di == 0:
                    return (
                        lambda t: (pos >= t) & (pos < 3),
                        lambda t: pos - t,
                        lambda u: pos >= u + 1,
                        lambda u: pos - 1 - u,
                        1,
                    )
                return (
                    lambda t: (pos <= 3 - t) & (pos > 0),
                    lambda t: pos + t,
                    lambda u: pos <= 2 - u,
                    lambda u: pos + 1 + u,
                    -1,
                )

            def mkrdma(di, step, blk, c):
                ro, nr = chunks_fn(jnp.clip(blk, 0, 3))[c]
                idx = (di, step, c) if nchunk > 1 else (di, step)
                return pltpu.make_async_remote_copy(
                    src_ref=out_ref.at[pl.ds(ro, nr), pl.ds(col0, _C)],
                    dst_ref=out_ref.at[pl.ds(ro, nr), pl.ds(col0, _C)],
                    send_sem=send_sems.at[idx],
                    recv_sem=recv_sems.at[idx],
                    device_id=dev(ops(di)[4]),
                    device_id_type=pl.DeviceIdType.MESH,
                )

            def send(t, c=None, extra=None):
                for di in range(2):
                    send_ok, send_blk, _, _, _ = ops(di)
                    cond = send_ok(t)
                    if extra is not None:
                        cond = cond & extra
                    for cc in range(nchunk) if c is None else [c]:
                        rdma = mkrdma(di, t, send_blk(t), cc)
                        pl.when(cond)(lambda rdma=rdma: rdma.start())
                        started.append((cond, rdma))

            def wait(u, c=None):
                for di in range(2):
                    _, _, recv_ok, recv_blk, _ = ops(di)
                    cond = recv_ok(u)
                    for cc in range(nchunk) if c is None else [c]:
                        rdma = mkrdma(di, u, recv_blk(u), cc)
                        pl.when(cond)(lambda rdma=rdma: rdma.wait_recv())

            return send, wait

        p1a = mkphase(
            "z", 0, lambda b: [((4 * Y + b) * _S, _S)], p1a_send, p1a_recv, 1
        )
        p1b = mkphase(
            "y", _C, lambda b: [((4 * b + Z) * _S, _S)], p1b_send, p1b_recv, 1
        )
        p2a = mkphase(
            "y", 0,
            lambda b: [((4 * b + c) * _S, _S) for c in range(4)],
            p2a_send, p2a_recv, 4,
        )
        p2b = mkphase(
            "z", _C,
            lambda b: [((4 * c + b) * _S, _S) for c in range(4)],
            p2b_send, p2b_recv, 4,
        )

        p1a[0](0)
        p1b[0](0)
        for c in range(4):
            p2a[0](0, c, extra=(c == Z))
            p2b[0](0, c, extra=(c == Y))
        for u in range(3):
            p1a[1](u)
            for c in range(4):
                p2a[0](0, c, extra=((c == Z - 1 - u) | (c == Z + 1 + u)))
            p1b[1](u)
            for c in range(4):
                p2b[0](0, c, extra=((c == Y - 1 - u) | (c == Y + 1 + u)))
            if u + 1 <= 2:
                p1a[0](u + 1)
                p1b[0](u + 1)
        for t in range(1, 4):
            for c in range(4):
                p2a[1](t - 1, c)
                if t <= 2:
                    p2a[0](t, c)
            for c in range(4):
                p2b[1](t - 1, c)
                if t <= 2:
                    p2b[0](t, c)

        xr.wait_send()
        for cond, rdma in started:
            pl.when(cond)(lambda rdma=rdma: rdma.wait_send())

    return pl.pallas_call(
        body,
        out_shape=jax.ShapeDtypeStruct((m, d), jnp.bfloat16),
        in_specs=[
            pl.BlockSpec(memory_space=pl.ANY),
            pl.BlockSpec(memory_space=pl.ANY),
            pl.BlockSpec(memory_space=pltpu.VMEM),
        ],
        out_specs=pl.BlockSpec(memory_space=pltpu.VMEM),
        scratch_shapes=[
            pltpu.VMEM((_S, d), jnp.float32),
            pltpu.VMEM((_S, d), jnp.bfloat16),
            pltpu.VMEM((_S, d), jnp.bfloat16),
            pltpu.VMEM((_S, d), jnp.float32),
            pltpu.SemaphoreType.DMA((2,)),
            pltpu.SemaphoreType.DMA,
            pltpu.SemaphoreType.DMA,
            pltpu.SemaphoreType.DMA((2, 3)),
            pltpu.SemaphoreType.DMA((2, 3)),
            pltpu.SemaphoreType.DMA((2, 3)),
            pltpu.SemaphoreType.DMA((2, 3)),
            pltpu.SemaphoreType.DMA((2, 3, 4)),
            pltpu.SemaphoreType.DMA((2, 3, 4)),
            pltpu.SemaphoreType.DMA((2, 3, 4)),
            pltpu.SemaphoreType.DMA((2, 3, 4)),
        ],
        compiler_params=pltpu.CompilerParams(
            collective_id=0, vmem_limit_bytes=60 * 1024 * 1024
        ),
    )(p, resid, g2)


# device time: 237987 ns/iter; 1.0549x vs baseline; 1.0549x over previous
import jax
import jax.numpy as jnp
from jax import lax
from jax.experimental import pallas as pl
from jax.experimental.pallas import tpu as pltpu

_S = 256
_C = 2048
_TILE = 128


def kernel(partial, resid, gamma):
    m, d = resid.shape
    p = partial.reshape(m, d)
    g2 = gamma.reshape(1, -1)

    def body(
        p_hbm, r_hbm, g_ref, out_ref,
        pst, pstb, ppr, rst,
        loc_sems, x_send, x_recv,
        p1a_send, p1a_recv, p1b_send, p1b_recv,
        p2a_send, p2a_recv, p2b_send, p2b_recv,
    ):
        X = lax.axis_index("x")
        Y = lax.axis_index("y")
        Z = lax.axis_index("z")
        s = 4 * Y + Z
        r0 = s * _S

        cp_p = pltpu.make_async_copy(
            p_hbm.at[pl.ds(r0, _S), :], pst, loc_sems.at[0]
        )
        cp_r = pltpu.make_async_copy(
            r_hbm.at[pl.ds(r0, _S), :], rst, loc_sems.at[1]
        )
        cp_p.start()
        cp_r.start()

        barrier = pltpu.get_barrier_semaphore()

        def sig(dev):
            pl.semaphore_signal(
                barrier, inc=1, device_id=dev,
                device_id_type=pl.DeviceIdType.MESH,
            )

        sig((1 - X, Y, Z))
        pl.when(Y > 0)(lambda: sig((X, Y - 1, Z)))
        pl.when(Y < 3)(lambda: sig((X, Y + 1, Z)))
        pl.when(Z > 0)(lambda: sig((X, Y, Z - 1)))
        pl.when(Z < 3)(lambda: sig((X, Y, Z + 1)))
        pl.semaphore_wait(barrier, 1)
        pl.when(Y > 0)(lambda: pl.semaphore_wait(barrier, 1))
        pl.when(Y < 3)(lambda: pl.semaphore_wait(barrier, 1))
        pl.when(Z > 0)(lambda: pl.semaphore_wait(barrier, 1))
        pl.when(Z < 3)(lambda: pl.semaphore_wait(barrier, 1))

        cp_p.wait()
        pstb[:, :] = pst[:, :].astype(jnp.bfloat16)
        xr = pltpu.make_async_remote_copy(
            src_ref=pstb,
            dst_ref=ppr,
            send_sem=x_send,
            recv_sem=x_recv,
            device_id=(1 - X, Y, Z),
            device_id_type=pl.DeviceIdType.MESH,
        )
        xr.start()
        xr.wait_recv()
        cp_r.wait()

        g = g_ref[:, :].astype(jnp.float32)
        for t in range(_S // _TILE):
            tr = pl.ds(t * _TILE, _TILE)
            y32 = (
                pst[tr, :]
                + ppr[tr, :].astype(jnp.float32)
                + rst[tr, :]
            )
            ms = jnp.mean(y32 * y32, axis=-1, keepdims=True)
            out_ref[pl.ds(r0 + t * _TILE, _TILE), :] = (
                y32 * lax.rsqrt(ms + 1e-6) * g
            ).astype(jnp.bfloat16)

        started = []

        def mkphase(axis, col0, chunks_fn, send_sems, recv_sems, nchunk):
            pos = Z if axis == "z" else Y

            def dev(off):
                if axis == "z":
                    return (X, Y, Z + off)
                return (X, Y + off, Z)

            def ops(di):
                if di == 0:
                    return (
                        lambda t: (pos >= t) & (pos < 3),
                        lambda t: pos - t,
                        lambda u: pos >= u + 1,
                        lambda u: pos - 1 - u,
                        1,
                    )
                return (
                    lambda t: (pos <= 3 - t) & (pos > 0),
                    lambda t: pos + t,
                    lambda u: pos <= 2 - u,
                    lambda u: pos + 1 + u,
                    -1,
                )

            def mkrdma(di, step, blk, c):
                ro, nr = chunks_fn(jnp.clip(blk, 0, 3))[c]
                idx = (di, step, c) if nchunk > 1 else (di, step)
                return pltpu.make_async_remote_copy(
                    src_ref=out_ref.at[pl.ds(ro, nr), pl.ds(col0, _C)],
                    dst_ref=out_ref.at[pl.ds(ro, nr), pl.ds(col0, _C)],
                    send_sem=send_sems.at[idx],
                    recv_sem=recv_sems.at[idx],
                    device_id=dev(ops(di)[4]),
                    device_id_type=pl.DeviceIdType.MESH,
                )

            def send(t, c=None, extra=None):
                for di in range(2):
                    send_ok, send_blk, _, _, _ = ops(di)
                    cond = send_ok(t)
                    if extra is not None:
                        cond = cond & extra
                    for cc in range(nchunk) if c is None else [c]:
                        rdma = mkrdma(di, t, send_blk(t), cc)
                        pl.when(cond)(lambda rdma=rdma: rdma.start())
                        started.append((cond, rdma))

            def wait(u, c=None, extra=None):
                for di in range(2):
                    _, _, recv_ok, recv_blk, _ = ops(di)
                    cond = recv_ok(u)
                    if extra is not None:
                        cond = cond & extra
                    for cc in range(nchunk) if c is None else [c]:
                        rdma = mkrdma(di, u, recv_blk(u), cc)
                        pl.when(cond)(lambda rdma=rdma: rdma.wait_recv())

            return send, wait

        p1a = mkphase(
            "z", 0, lambda b: [((4 * Y + b) * _S, _S)], p1a_send, p1a_recv, 1
        )
        p1b = mkphase(
            "y", _C, lambda b: [((4 * b + Z) * _S, _S)], p1b_send, p1b_recv, 1
        )
        p2a = mkphase(
            "y", 0,
            lambda b: [((4 * b + c) * _S, _S) for c in range(4)],
            p2a_send, p2a_recv, 4,
        )
        p2b = mkphase(
            "z", _C,
            lambda b: [((4 * c + b) * _S, _S) for c in range(4)],
            p2b_send, p2b_recv, 4,
        )

        p1a[0](0)
        p1b[0](0)
        for c in range(4):
            p2a[0](0, c, extra=(c == Z))
            p2b[0](0, c, extra=(c == Y))
        for u in range(4):
            if u <= 2:
                p1a[1](u)
                for c in range(4):
                    p2a[0](0, c, extra=((c == Z - 1 - u) | (c == Z + 1 + u)))
                p1b[1](u)
                for c in range(4):
                    p2b[0](0, c, extra=((c == Y - 1 - u) | (c == Y + 1 + u)))
                if u + 1 <= 2:
                    p1a[0](u + 1)
                    p1b[0](u + 1)
            for c in range(4):
                cz = (c == Z - u) | (c == Z + u)
                p2a[1](0, c, extra=cz)
                p2a[0](1, c, extra=cz)
                cy = (c == Y - u) | (c == Y + u)
                p2b[1](0, c, extra=cy)
                p2b[0](1, c, extra=cy)
        for c in range(4):
            p2a[1](1, c)
            p2a[0](2, c)
            p2b[1](1, c)
            p2b[0](2, c)
        for c in range(4):
            p2a[1](2, c)
            p2b[1](2, c)

        xr.wait_send()
        for cond, rdma in started:
            pl.when(cond)(lambda rdma=rdma: rdma.wait_send())

    return pl.pallas_call(
        body,
        out_shape=jax.ShapeDtypeStruct((m, d), jnp.bfloat16),
        in_specs=[
            pl.BlockSpec(memory_space=pl.ANY),
            pl.BlockSpec(memory_space=pl.ANY),
            pl.BlockSpec(memory_space=pltpu.VMEM),
        ],
        out_specs=pl.BlockSpec(memory_space=pltpu.VMEM),
        scratch_shapes=[
            pltpu.VMEM((_S, d), jnp.float32),
            pltpu.VMEM((_S, d), jnp.bfloat16),
            pltpu.VMEM((_S, d), jnp.bfloat16),
            pltpu.VMEM((_S, d), jnp.float32),
            pltpu.SemaphoreType.DMA((2,)),
            pltpu.SemaphoreType.DMA,
            pltpu.SemaphoreType.DMA,
            pltpu.SemaphoreType.DMA((2, 3)),
            pltpu.SemaphoreType.DMA((2, 3)),
            pltpu.SemaphoreType.DMA((2, 3)),
            pltpu.SemaphoreType.DMA((2, 3)),
            pltpu.SemaphoreType.DMA((2, 3, 4)),
            pltpu.SemaphoreType.DMA((2, 3, 4)),
            pltpu.SemaphoreType.DMA((2, 3, 4)),
            pltpu.SemaphoreType.DMA((2, 3, 4)),
        ],
        compiler_params=pltpu.CompilerParams(
            collective_id=0, vmem_limit_bytes=60 * 1024 * 1024
        ),
    )(p, resid, g2)


# device time: 235211 ns/iter; 1.0673x vs baseline; 1.0118x over previous
import jax
import jax.numpy as jnp
from jax import lax
from jax.experimental import pallas as pl
from jax.experimental.pallas import tpu as pltpu

_S = 256
_C = 2048
_TILE = 128


def kernel(partial, resid, gamma):
    m, d = resid.shape
    p = partial.reshape(m, d)
    g2 = gamma.reshape(1, -1)

    def body(
        p_hbm, r_hbm, g_ref, out_ref,
        pst, pstb, ppr, rst,
        loc_sems, x_send, x_recv,
        p1a_send, p1a_recv, p1b_send, p1b_recv,
        p2a_send, p2a_recv, p2b_send, p2b_recv,
    ):
        X = lax.axis_index("x")
        Y = lax.axis_index("y")
        Z = lax.axis_index("z")
        s = 4 * Y + Z
        r0 = s * _S

        cp_p = pltpu.make_async_copy(
            p_hbm.at[pl.ds(r0, _S), :], pst, loc_sems.at[0]
        )
        cp_r = pltpu.make_async_copy(
            r_hbm.at[pl.ds(r0, _S), :], rst, loc_sems.at[1]
        )
        cp_p.start()
        cp_r.start()

        barrier = pltpu.get_barrier_semaphore()

        def sig(dev):
            pl.semaphore_signal(
                barrier, inc=1, device_id=dev,
                device_id_type=pl.DeviceIdType.MESH,
            )

        sig((1 - X, Y, Z))
        pl.when(Y > 0)(lambda: sig((X, Y - 1, Z)))
        pl.when(Y < 3)(lambda: sig((X, Y + 1, Z)))
        pl.when(Z > 0)(lambda: sig((X, Y, Z - 1)))
        pl.when(Z < 3)(lambda: sig((X, Y, Z + 1)))
        pl.semaphore_wait(barrier, 1)
        pl.when(Y > 0)(lambda: pl.semaphore_wait(barrier, 1))
        pl.when(Y < 3)(lambda: pl.semaphore_wait(barrier, 1))
        pl.when(Z > 0)(lambda: pl.semaphore_wait(barrier, 1))
        pl.when(Z < 3)(lambda: pl.semaphore_wait(barrier, 1))

        cp_p.wait()
        pstb[:, :] = pst[:, :].astype(jnp.bfloat16)
        xrs = []
        for h in range(2):
            hr = pl.ds(h * _TILE, _TILE)
            xr = pltpu.make_async_remote_copy(
                src_ref=pstb.at[hr, :],
                dst_ref=ppr.at[hr, :],
                send_sem=x_send.at[h],
                recv_sem=x_recv.at[h],
                device_id=(1 - X, Y, Z),
                device_id_type=pl.DeviceIdType.MESH,
            )
            xr.start()
            xrs.append(xr)
        cp_r.wait()

        g = g_ref[:, :].astype(jnp.float32)
        for t in range(_S // _TILE):
            tr = pl.ds(t * _TILE, _TILE)
            xrs[t].wait_recv()
            y32 = (
                pst[tr, :]
                + ppr[tr, :].astype(jnp.float32)
                + rst[tr, :]
            )
            ms = jnp.mean(y32 * y32, axis=-1, keepdims=True)
            out_ref[pl.ds(r0 + t * _TILE, _TILE), :] = (
                y32 * lax.rsqrt(ms + 1e-6) * g
            ).astype(jnp.bfloat16)

        started = []

        def mkphase(axis, col0, chunks_fn, send_sems, recv_sems, nchunk):
            pos = Z if axis == "z" else Y

            def dev(off):
                if axis == "z":
                    return (X, Y, Z + off)
                return (X, Y + off, Z)

            def ops(di):
                if di == 0:
                    return (
                        lambda t: (pos >= t) & (pos < 3),
                        lambda t: pos - t,
                        lambda u: pos >= u + 1,
                        lambda u: pos - 1 - u,
                        1,
                    )
                return (
                    lambda t: (pos <= 3 - t) & (pos > 0),
                    lambda t: pos + t,
                    lambda u: pos <= 2 - u,
                    lambda u: pos + 1 + u,
                    -1,
                )

            def mkrdma(di, step, blk, c):
                ro, nr = chunks_fn(jnp.clip(blk, 0, 3))[c]
                idx = (di, step, c) if nchunk > 1 else (di, step)
                return pltpu.make_async_remote_copy(
                    src_ref=out_ref.at[pl.ds(ro, nr), pl.ds(col0, _C)],
                    dst_ref=out_ref.at[pl.ds(ro, nr), pl.ds(col0, _C)],
                    send_sem=send_sems.at[idx],
                    recv_sem=recv_sems.at[idx],
                    device_id=dev(ops(di)[4]),
                    device_id_type=pl.DeviceIdType.MESH,
                )

            def send(t, c=None, extra=None):
                for di in range(2):
                    send_ok, send_blk, _, _, _ = ops(di)
                    cond = send_ok(t)
                    if extra is not None:
                        cond = cond & extra
                    for cc in range(nchunk) if c is None else [c]:
                        rdma = mkrdma(di, t, send_blk(t), cc)
                        pl.when(cond)(lambda rdma=rdma: rdma.start())
                        started.append((cond, rdma))

            def wait(u, c=None, extra=None):
                for di in range(2):
                    _, _, recv_ok, recv_blk, _ = ops(di)
                    cond = recv_ok(u)
                    if extra is not None:
                        cond = cond & extra
                    for cc in range(nchunk) if c is None else [c]:
                        rdma = mkrdma(di, u, recv_blk(u), cc)
                        pl.when(cond)(lambda rdma=rdma: rdma.wait_recv())

            return send, wait

        p1a = mkphase(
            "z", 0, lambda b: [((4 * Y + b) * _S, _S)], p1a_send, p1a_recv, 1
        )
        p1b = mkphase(
            "y", _C, lambda b: [((4 * b + Z) * _S, _S)], p1b_send, p1b_recv, 1
        )
        p2a = mkphase(
            "y", 0,
            lambda b: [((4 * b + c) * _S, _S) for c in range(4)],
            p2a_send, p2a_recv, 4,
        )
        p2b = mkphase(
            "z", _C,
            lambda b: [((4 * c + b) * _S, _S) for c in range(4)],
            p2b_send, p2b_recv, 4,
        )

        p1a[0](0)
        p1b[0](0)
        for c in range(4):
            p2a[0](0, c, extra=(c == Z))
            p2b[0](0, c, extra=(c == Y))
        for u in range(4):
            if u <= 2:
                p1a[1](u)
                p1b[1](u)
                if u + 1 <= 2:
                    p1a[0](u + 1)
                    p1b[0](u + 1)
                for c in range(4):
                    p2a[0](0, c, extra=((c == Z - 1 - u) | (c == Z + 1 + u)))
                for c in range(4):
                    p2b[0](0, c, extra=((c == Y - 1 - u) | (c == Y + 1 + u)))
            for c in range(4):
                cz = (c == Z - u) | (c == Z + u)
                p2a[1](0, c, extra=cz)
                p2a[0](1, c, extra=cz)
                cy = (c == Y - u) | (c == Y + u)
                p2b[1](0, c, extra=cy)
                p2b[0](1, c, extra=cy)
        for c in range(4):
            p2a[1](1, c)
            p2a[0](2, c)
            p2b[1](1, c)
            p2b[0](2, c)
        for c in range(4):
            p2a[1](2, c)
            p2b[1](2, c)

        for xr in xrs:
            xr.wait_send()
        for cond, rdma in started:
            pl.when(cond)(lambda rdma=rdma: rdma.wait_send())

    return pl.pallas_call(
        body,
        out_shape=jax.ShapeDtypeStruct((m, d), jnp.bfloat16),
        in_specs=[
            pl.BlockSpec(memory_space=pl.ANY),
            pl.BlockSpec(memory_space=pl.ANY),
            pl.BlockSpec(memory_space=pltpu.VMEM),
        ],
        out_specs=pl.BlockSpec(memory_space=pltpu.VMEM),
        scratch_shapes=[
            pltpu.VMEM((_S, d), jnp.float32),
            pltpu.VMEM((_S, d), jnp.bfloat16),
            pltpu.VMEM((_S, d), jnp.bfloat16),
            pltpu.VMEM((_S, d), jnp.float32),
            pltpu.SemaphoreType.DMA((2,)),
            pltpu.SemaphoreType.DMA((2,)),
            pltpu.SemaphoreType.DMA((2,)),
            pltpu.SemaphoreType.DMA((2, 3)),
            pltpu.SemaphoreType.DMA((2, 3)),
            pltpu.SemaphoreType.DMA((2, 3)),
            pltpu.SemaphoreType.DMA((2, 3)),
            pltpu.SemaphoreType.DMA((2, 3, 4)),
            pltpu.SemaphoreType.DMA((2, 3, 4)),
            pltpu.SemaphoreType.DMA((2, 3, 4)),
            pltpu.SemaphoreType.DMA((2, 3, 4)),
        ],
        compiler_params=pltpu.CompilerParams(
            collective_id=0, vmem_limit_bytes=60 * 1024 * 1024
        ),
    )(p, resid, g2)
